# baseline (speedup 1.0000x reference)
import functools

import jax
import jax.numpy as jnp
from jax import lax
from jax.experimental import pallas as pl
from jax.experimental.pallas import tpu as pltpu
from jax.experimental.pallas import tpu_sc as plsc

D = 128
NT = 10000
E = 160000
OUT = 16

_NSUB = 16
_EPS = E // _NSUB
_NTP = 10240
_STRIDE = _NTP // _NSUB


def _tc_body(edge, aWv, abv, aWo, abo, tWv, tbv, tWo, tbo,
             s1Wl, s1bl, s1Wr, out_w, out_b, out_ref, dst_ref):
    dst_ref[...] = edge[1, :]
    va = jnp.sum(aWv[...], axis=0, keepdims=True) + abv[...]
    u = jnp.dot(va, aWo[...], preferred_element_type=jnp.float32) + abo[...]
    vt = jnp.sum(tWv[...], axis=0, keepdims=True) + tbv[...]
    ct = jnp.dot(vt, tWo[...], preferred_element_type=jnp.float32) + tbo[...]
    c = ct[0:1, 0:1]
    base = s1bl[...] + c * s1Wr[...]
    row1 = jnp.dot(u, s1Wl[...], preferred_element_type=jnp.float32) + base
    rows = jnp.concatenate([base, row1], axis=0)
    logits = jnp.dot(rows, out_w[...],
                     preferred_element_type=jnp.float32) + out_b[...]
    m = jnp.max(logits, axis=1, keepdims=True)
    e = jnp.exp(logits - m)
    p = e / jnp.sum(e, axis=1, keepdims=True)
    out_ref[0:1, :] = p[0:1, :]
    out_ref[1:2, :] = p[1:2, :] - p[0:1, :]


def _tc_pd(edge, aWv, abv, aWo, abo, tWv, tbv, tWo, tbo, s1Wl, s1bl, s1Wr,
           out_w, out_b):
    r1 = lambda v: jnp.reshape(v, (1, -1))
    return pl.pallas_call(
        _tc_body,
        out_shape=[jax.ShapeDtypeStruct((2, OUT), jnp.float32),
                   jax.ShapeDtypeStruct((E,), jnp.int32)],
    )(edge, aWv, r1(abv), aWo, r1(abo), tWv, r1(tbv), tWo, r1(tbo),
      s1Wl, r1(s1bl), s1Wr, out_w, r1(out_b))


def _sc_body(dst_hbm, pd_hbm, out_hbm, idx_v, flags_v, pd_v, strip_v,
             outblk_v, shared, sem):
    t = lax.axis_index("s")

    cp = pltpu.make_async_copy(dst_hbm.at[pl.ds(t * _EPS, _EPS)], idx_v, sem)
    cp.start()
    pltpu.sync_copy(pd_hbm, pd_v)
    p0 = pd_v[0, :]
    dv = pd_v[1, :]

    zero16 = jnp.zeros((16,), jnp.float32)
    ones16 = jnp.ones((16,), jnp.float32)

    with jax.named_scope("ph_zero"):
        @plsc.parallel_loop(0, _NTP // 16, unroll=8)
        def _zero(i):
            flags_v[pl.ds(i * 16, 16)] = zero16
    with jax.named_scope("ph_idxwait"):
        cp.wait()

    with jax.named_scope("ph_scatter"):
        @plsc.parallel_loop(0, _EPS // 16, unroll=8)
        def _scatter(i):
            iv = idx_v[pl.ds(i * 16, 16)]
            plsc.store_scatter(flags_v, [iv], ones16)

    with jax.named_scope("ph_publish"):
        pltpu.sync_copy(flags_v, shared.at[t])
        plsc.subcore_barrier()

    base = t * _STRIDE
    with jax.named_scope("ph_gather"):
        pltpu.sync_copy(shared.at[:, pl.ds(base, _STRIDE)], strip_v)

    with jax.named_scope("ph_assemble"):
        _assemble(strip_v, outblk_v, p0, dv)

    cid = lax.axis_index("c")
    tail = NT - (_NSUB - 1) * _STRIDE
    half = _STRIDE // 2
    off = cid * half
    with jax.named_scope("ph_outwrite"):
        @pl.when(t < _NSUB - 1)
        def _():
            pltpu.sync_copy(
                outblk_v.at[pl.ds(off * OUT, half * OUT)],
                out_hbm.at[pl.ds((base + off) * OUT, half * OUT)])
        @pl.when((t == _NSUB - 1) & (cid == 0))
        def _():
            pltpu.sync_copy(
                outblk_v.at[pl.ds(0, half * OUT)],
                out_hbm.at[pl.ds(base * OUT, half * OUT)])
        @pl.when((t == _NSUB - 1) & (cid == 1))
        def _():
            pltpu.sync_copy(
                outblk_v.at[pl.ds(half * OUT, (tail - half) * OUT)],
                out_hbm.at[pl.ds((base + half) * OUT, (tail - half) * OUT)])


def _assemble(strip_v, outblk_v, p0, dv):
    @plsc.parallel_loop(0, _STRIDE // 16, unroll=2)
    def _chunk(c):
        rows = [strip_v[r, pl.ds(c * 16, 16)] for r in range(_NSUB)]
        while len(rows) > 1:
            rows = [a + b for a, b in zip(rows[::2], rows[1::2])]
        ind = jnp.minimum(rows[0], 1.0)
        for n in range(16):
            outblk_v[pl.ds((c * 16 + n) * OUT, OUT)] = p0 + ind[n] * dv


def _sc_scatter_build():
    mesh = plsc.VectorSubcoreMesh(core_axis_name="c", subcore_axis_name="s")
    return pl.kernel(
        _sc_body,
        mesh=mesh,
        compiler_params=pltpu.CompilerParams(needs_layout_passes=False),
        out_type=jax.ShapeDtypeStruct((NT * OUT,), jnp.float32),
        scratch_types=[
            pltpu.VMEM((_EPS,), jnp.int32),
            pltpu.VMEM((_NTP,), jnp.float32),
            pltpu.VMEM((2, OUT), jnp.float32),
            pltpu.VMEM((_NSUB, _STRIDE), jnp.float32),
            pltpu.VMEM((_STRIDE * OUT,), jnp.float32),
            pltpu.VMEM_SHARED((_NSUB, _NTP), jnp.float32),
            pltpu.SemaphoreType.DMA,
        ],
    )


def kernel(x_a_cat, x_a_num, x_t, edge_a2t, edge_t2a, emb_cat, num_w, num_b,
           aWq, aWk, aWv, aWo, abq, abk, abv, abo,
           tWq, tWk, tWv, tWo, tbq, tbk, tbv, tbo,
           s1Wl, s1bl, s1Wr, s2Wl, s2bl, s2Wr, out_w, out_b):
    pd, dst = _tc_pd(edge_a2t.astype(jnp.int32), aWv, abv, aWo, abo,
                     tWv, tbv, tWo, tbo, s1Wl, s1bl, s1Wr, out_w, out_b)
    return jnp.reshape(_sc_scatter_build()(dst, pd), (NT, OUT))


# speedup vs baseline: 1.0098x; 1.0098x over previous
import functools

import jax
import jax.numpy as jnp
from jax import lax
from jax.experimental import pallas as pl
from jax.experimental.pallas import tpu as pltpu
from jax.experimental.pallas import tpu_sc as plsc

D = 128
NT = 10000
E = 160000
OUT = 16

_NSUB = 16
_EPS = E // _NSUB
_NTP = 10240
_STRIDE = _NTP // _NSUB


def _tc_body(edge, aWv, abv, aWo, abo, tWv, tbv, tWo, tbo,
             s1Wl, s1bl, s1Wr, out_w, out_b, out_ref, dst_ref):
    dst_ref[...] = edge[1, :]
    va = jnp.sum(aWv[...], axis=0, keepdims=True) + abv[...]
    u = jnp.dot(va, aWo[...], preferred_element_type=jnp.float32) + abo[...]
    vt = jnp.sum(tWv[...], axis=0, keepdims=True) + tbv[...]
    ct = jnp.dot(vt, tWo[...], preferred_element_type=jnp.float32) + tbo[...]
    c = ct[0:1, 0:1]
    base = s1bl[...] + c * s1Wr[...]
    row1 = jnp.dot(u, s1Wl[...], preferred_element_type=jnp.float32) + base
    rows = jnp.concatenate([base, row1], axis=0)
    logits = jnp.dot(rows, out_w[...],
                     preferred_element_type=jnp.float32) + out_b[...]
    m = jnp.max(logits, axis=1, keepdims=True)
    e = jnp.exp(logits - m)
    p = e / jnp.sum(e, axis=1, keepdims=True)
    out_ref[0:1, :] = p[0:1, :]
    out_ref[1:2, :] = p[1:2, :] - p[0:1, :]


def _tc_pd(edge, aWv, abv, aWo, abo, tWv, tbv, tWo, tbo, s1Wl, s1bl, s1Wr,
           out_w, out_b):
    r1 = lambda v: jnp.reshape(v, (1, -1))
    return pl.pallas_call(
        _tc_body,
        out_shape=[jax.ShapeDtypeStruct((2, OUT), jnp.float32),
                   jax.ShapeDtypeStruct((E,), jnp.int32)],
    )(edge, aWv, r1(abv), aWo, r1(abo), tWv, r1(tbv), tWo, r1(tbo),
      s1Wl, r1(s1bl), s1Wr, out_w, r1(out_b))


def _sc_body(dst_hbm, pd_hbm, out_hbm, idx_v, flags_v, pd_v, strip_v,
             outblk_v, shared, sem):
    t = lax.axis_index("s")

    cp = pltpu.make_async_copy(dst_hbm.at[pl.ds(t * _EPS, _EPS)], idx_v, sem)
    cp.start()
    pltpu.sync_copy(pd_hbm, pd_v)
    p0 = pd_v[0, :]
    dv = pd_v[1, :]

    zero16 = jnp.zeros((16,), jnp.float32)
    ones16 = jnp.ones((16,), jnp.float32)

    with jax.named_scope("ph_zero"):
        @plsc.parallel_loop(0, _NTP // 16, unroll=8)
        def _zero(i):
            flags_v[pl.ds(i * 16, 16)] = zero16
    with jax.named_scope("ph_idxwait"):
        cp.wait()

    with jax.named_scope("ph_scatter"):
        @plsc.parallel_loop(0, _EPS // 16, unroll=8)
        def _scatter(i):
            iv = idx_v[pl.ds(i * 16, 16)]
            plsc.store_scatter(flags_v, [iv], ones16)

    with jax.named_scope("ph_publish"):
        pltpu.sync_copy(flags_v, shared.at[t])
        plsc.subcore_barrier()

    base = t * _STRIDE
    with jax.named_scope("ph_gather"):
        pltpu.sync_copy(shared.at[:, pl.ds(base, _STRIDE)], strip_v)

    with jax.named_scope("ph_assemble"):
        _assemble(strip_v, outblk_v, p0, dv)

    cid = lax.axis_index("c")
    tail = NT - (_NSUB - 1) * _STRIDE
    half = _STRIDE // 2
    off = cid * half
    with jax.named_scope("ph_outwrite"):
        @pl.when(t < _NSUB - 1)
        def _():
            pltpu.sync_copy(outblk_v.at[pl.ds(off, half), :],
                            out_hbm.at[pl.ds(base + off, half), :])
        @pl.when((t == _NSUB - 1) & (cid == 0))
        def _():
            pltpu.sync_copy(outblk_v.at[pl.ds(0, half), :],
                            out_hbm.at[pl.ds(base, half), :])
        @pl.when((t == _NSUB - 1) & (cid == 1))
        def _():
            pltpu.sync_copy(outblk_v.at[pl.ds(half, tail - half), :],
                            out_hbm.at[pl.ds(base + half, tail - half), :])


def _assemble(strip_v, outblk_v, p0, dv):
    @plsc.parallel_loop(0, _STRIDE // 16, unroll=2)
    def _chunk(c):
        rows = [strip_v[r, pl.ds(c * 16, 16)] for r in range(_NSUB)]
        while len(rows) > 1:
            rows = [a + b for a, b in zip(rows[::2], rows[1::2])]
        ind = jnp.minimum(rows[0], 1.0)
        for n in range(16):
            outblk_v[c * 16 + n, :] = p0 + ind[n] * dv


def _sc_scatter_build():
    mesh = plsc.VectorSubcoreMesh(core_axis_name="c", subcore_axis_name="s")
    return pl.kernel(
        _sc_body,
        mesh=mesh,
        compiler_params=pltpu.CompilerParams(needs_layout_passes=False),
        out_type=jax.ShapeDtypeStruct((NT, OUT), jnp.float32),
        scratch_types=[
            pltpu.VMEM((_EPS,), jnp.int32),
            pltpu.VMEM((_NTP,), jnp.float32),
            pltpu.VMEM((2, OUT), jnp.float32),
            pltpu.VMEM((_NSUB, _STRIDE), jnp.float32),
            pltpu.VMEM((_STRIDE, OUT), jnp.float32),
            pltpu.VMEM_SHARED((_NSUB, _NTP), jnp.float32),
            pltpu.SemaphoreType.DMA,
        ],
    )


def kernel(x_a_cat, x_a_num, x_t, edge_a2t, edge_t2a, emb_cat, num_w, num_b,
           aWq, aWk, aWv, aWo, abq, abk, abv, abo,
           tWq, tWk, tWv, tWo, tbq, tbk, tbv, tbo,
           s1Wl, s1bl, s1Wr, s2Wl, s2bl, s2Wr, out_w, out_b):
    pd, dst = _tc_pd(edge_a2t.astype(jnp.int32), aWv, abv, aWo, abo,
                     tWv, tbv, tWo, tbo, s1Wl, s1bl, s1Wr, out_w, out_b)
    return _sc_scatter_build()(dst, pd)
